# Initial kernel scaffold; baseline (speedup 1.0000x reference)
import jax
import jax.numpy as jnp
from jax.experimental import pallas as pl


def kernel(x, table):
    raise NotImplementedError("write your pallas kernel here")


# speedup vs baseline: 1.5747x; 1.5747x over previous
import functools

import jax
import jax.numpy as jnp
from jax import lax
from jax.experimental import pallas as pl
from jax.experimental.pallas import tpu as pltpu
from jax.experimental.pallas import tpu_sc as plsc

_NC = 2
_NS = 16
_NW = _NC * _NS


def _pick_chunk(b_per_w: int) -> int:
    for c in range(min(2048, b_per_w), 0, -1):
        if b_per_w % c == 0:
            return c
    return b_per_w


@functools.lru_cache(maxsize=None)
def _make_gather(num_rows: int, dim: int, b_total: int):
    assert b_total % _NW == 0
    b_per_w = b_total // _NW
    chunk = _pick_chunk(b_per_w)
    n_chunks = b_per_w // chunk

    mesh = plsc.VectorSubcoreMesh(core_axis_name="c", subcore_axis_name="s")

    @functools.partial(
        pl.kernel,
        mesh=mesh,
        out_type=jax.ShapeDtypeStruct((b_total, dim), jnp.float32),
        compiler_params=pltpu.CompilerParams(use_tc_tiling_on_sc=False),
        scratch_types=[
            pltpu.VMEM((n_chunks, chunk), jnp.int32),
            pltpu.VMEM((chunk, dim), jnp.float32),
            pltpu.VMEM((chunk, dim), jnp.float32),
            pltpu.SemaphoreType.DMA,
            pltpu.SemaphoreType.DMA,
            pltpu.SemaphoreType.DMA,
            pltpu.SemaphoreType.DMA,
        ],
    )
    def gather_kernel(idx_hbm, table_hbm, out_hbm, idx_v, rows0, rows1,
                      gsem0, gsem1, osem0, osem1):
        wid = lax.axis_index("s") * _NC + lax.axis_index("c")
        base = wid * b_per_w
        pltpu.sync_copy(idx_hbm.at[wid], idx_v)

        bufs = (rows0, rows1)
        gsems = (gsem0, gsem1)
        osems = (osem0, osem1)

        def start_gather(j, b):
            return pltpu.async_copy(table_hbm.at[idx_v.at[j]], bufs[b], gsems[b])

        def start_put(j, b):
            return pltpu.async_copy(
                bufs[b], out_hbm.at[pl.ds(base + j * chunk, chunk)], osems[b])

        out_h = [None, None]
        pending = [False, False]
        g_cur = start_gather(0, 0)
        for j in range(n_chunks):
            b = j & 1
            nb = 1 - b
            g_nxt = None
            if j + 1 < n_chunks:
                if pending[nb]:
                    out_h[nb].wait()
                    pending[nb] = False
                g_nxt = start_gather(j + 1, nb)
            g_cur.wait()
            out_h[b] = start_put(j, b)
            pending[b] = True
            g_cur = g_nxt
        for b in (0, 1):
            if pending[b]:
                out_h[b].wait()

    return gather_kernel, b_per_w, n_chunks, chunk


def kernel(x, table):
    batch, fields = x.shape
    num_rows, dim = table.shape
    b_total = batch * fields
    gather_kernel, _, n_chunks, chunk = _make_gather(num_rows, dim, b_total)
    idx = x.reshape(_NW, n_chunks, chunk).astype(jnp.int32)
    out = gather_kernel(idx, table)
    return out.reshape(batch, fields, dim)


# speedup vs baseline: 1.5819x; 1.0046x over previous
import functools

import jax
import jax.numpy as jnp
from jax import lax
from jax.experimental import pallas as pl
from jax.experimental.pallas import tpu as pltpu
from jax.experimental.pallas import tpu_sc as plsc

_NC = 2
_NS = 16
_NW = _NC * _NS


_NBUF = 4
_DEPTH = 3


_TILESPMEM_WORDS = 126976


def _pick_chunk(b_per_w: int, dim: int, fields: int) -> int:
    budget = (_TILESPMEM_WORDS - b_per_w) // (_NBUF * dim)
    for c in range(min(budget, b_per_w), 0, -1):
        if b_per_w % c == 0 and c % fields == 0:
            return c
    return b_per_w


@functools.lru_cache(maxsize=None)
def _make_gather(num_rows: int, dim: int, batch: int, fields: int):
    b_total = batch * fields
    assert b_total % _NW == 0
    b_per_w = b_total // _NW
    assert b_per_w % fields == 0
    chunk = _pick_chunk(b_per_w, dim, fields)
    n_chunks = b_per_w // chunk
    rows_per_chunk = chunk // fields

    mesh = plsc.VectorSubcoreMesh(core_axis_name="c", subcore_axis_name="s")

    @functools.partial(
        pl.kernel,
        mesh=mesh,
        out_type=jax.ShapeDtypeStruct((b_total, dim), jnp.float32),
        compiler_params=pltpu.CompilerParams(use_tc_tiling_on_sc=False),
        scratch_types=(
            [pltpu.VMEM((n_chunks, chunk), jnp.int32)]
            + [pltpu.VMEM((chunk, dim), jnp.float32) for _ in range(_NBUF)]
            + [pltpu.SemaphoreType.DMA for _ in range(2 * _NBUF)]
        ),
    )
    def gather_kernel(idx_hbm, table_hbm, out_hbm, idx_v, *bufs_and_sems):
        bufs = bufs_and_sems[:_NBUF]
        gsems = bufs_and_sems[_NBUF:2 * _NBUF]
        osems = bufs_and_sems[2 * _NBUF:]
        wid = lax.axis_index("s") * _NC + lax.axis_index("c")
        base = wid * b_per_w
        pltpu.sync_copy(idx_hbm.at[wid], idx_v)

        def start_gather(j, b):
            return pltpu.async_copy(table_hbm.at[idx_v.at[j]], bufs[b], gsems[b])

        def start_put(j, b):
            return pltpu.async_copy(
                bufs[b], out_hbm.at[pl.ds(base + j * chunk, chunk)], osems[b])

        gh = [None] * _NBUF
        oh = [None] * _NBUF
        opend = [False] * _NBUF
        depth = min(_DEPTH, n_chunks)
        for j in range(depth):
            gh[j % _NBUF] = start_gather(j, j % _NBUF)
        for j in range(n_chunks):
            b = j % _NBUF
            nj = j + depth
            if nj < n_chunks:
                nb = nj % _NBUF
                if opend[nb]:
                    oh[nb].wait()
                    opend[nb] = False
                gh[nb] = start_gather(nj, nb)
            gh[b].wait()
            oh[b] = start_put(j, b)
            opend[b] = True
        for b in range(_NBUF):
            if opend[b]:
                oh[b].wait()

    return gather_kernel, b_per_w, n_chunks, chunk


def kernel(x, table):
    batch, fields = x.shape
    num_rows, dim = table.shape
    gather_kernel, _, n_chunks, chunk = _make_gather(num_rows, dim, batch, fields)
    idx = x.reshape(_NW, n_chunks, chunk).astype(jnp.int32)
    out = gather_kernel(idx, table)
    return out.reshape(batch, fields, dim)
